# Initial kernel scaffold; baseline (speedup 1.0000x reference)
import jax
import jax.numpy as jnp
from jax.experimental import pallas as pl


def kernel(x, edge_index, time_emb, batch, W_gat, att_src, att_dst, bias_gat, W_time, b_time, ln_gamma, ln_beta):
    raise NotImplementedError("write your pallas kernel here")


# speedup vs baseline: 31.7440x; 31.7440x over previous
import jax
import jax.numpy as jnp
from jax import lax
from jax.experimental import pallas as pl
from jax.experimental.pallas import tpu as pltpu, tpu_sc as plsc

N, E, DIN, DOUT, DTIME, H, B = 10000, 320000, 128, 128, 256, 4, 64
N2 = 10240
R = 1024
W = 128
NT = 16
WIN = 160
EPT = WIN * W
EP = NT * EPT
RPT = N2 // NT


def _prep_body(x_ref, wg_ref, asw_ref, adw_ref, xh_ref, as_ref, ad_ref):
    xh = jnp.dot(x_ref[...], wg_ref[...], preferred_element_type=jnp.float32)
    xh_ref[0] = xh
    as_ref[0, 0] = jnp.sum(xh * asw_ref[0], axis=1)
    ad_ref[0, 0] = jnp.sum(xh * adw_ref[0], axis=1)


def _prep(x_pad, W_gat, att_src, att_dst):
    return pl.pallas_call(
        _prep_body,
        grid=(H, N2 // R),
        in_specs=[
            pl.BlockSpec((R, DIN), lambda h, r: (r, 0)),
            pl.BlockSpec((DIN, DOUT), lambda h, r: (0, h)),
            pl.BlockSpec((1, 1, DOUT), lambda h, r: (h, 0, 0)),
            pl.BlockSpec((1, 1, DOUT), lambda h, r: (h, 0, 0)),
        ],
        out_specs=[
            pl.BlockSpec((1, R, DOUT), lambda h, r: (h, r, 0)),
            pl.BlockSpec((1, 1, R), lambda h, r: (h, 0, r)),
            pl.BlockSpec((1, 1, R), lambda h, r: (h, 0, r)),
        ],
        out_shape=[
            jax.ShapeDtypeStruct((H, N2, DOUT), jnp.float32),
            jax.ShapeDtypeStruct((H, 1, N2), jnp.float32),
            jax.ShapeDtypeStruct((H, 1, N2), jnp.float32),
        ],
    )(x_pad, W_gat, att_src, att_dst)


def _sc_body(xh_hbm, tab_hbm, edges_hbm,
             num_hbm, den_hbm,
             num_acc, den_acc, tab_v, zden_v,
             ed0, ed1, ed2, ed3, wb0, wb1, rw0, rw1,
             is0, is1, is2, is3, gs0, gs1, ss0, ss1, ds0, ds1):
    c = lax.axis_index("c")
    s = lax.axis_index("s")
    ED = [ed0, ed1, ed2, ed3]
    IS = [is0, is1, is2, is3]
    WB = [wb0, wb1]
    RW = [rw0, rw1]
    GS = [gs0, gs1]
    SS = [ss0, ss1]
    DS = [ds0, ds1]
    base_r = s * RPT

    def _zero_rw0(i, _):
        for k in range(8):
            rw0[i, pl.ds(k * 16, 16)] = jnp.zeros((16,), jnp.float32)
        return 0

    def _zden(i, _):
        zden_v[pl.ds(i * 16, 16)] = jnp.zeros((16,), jnp.float32)
        return 0
    lax.fori_loop(0, RPT // 16, _zden, 0)
    lax.fori_loop(0, W, _zero_rw0, 0)

    def _head(j, _):
        h = c * 2 + j
        pltpu.sync_copy(tab_hbm.at[h, 0], tab_v)

        def _zacc(k, _):
            pltpu.sync_copy(rw0, num_acc.at[pl.ds(base_r + k * W, W)])
            return 0
        lax.fori_loop(0, RPT // W, _zacc, 0)
        pltpu.sync_copy(zden_v, den_acc.at[pl.ds(base_r, RPT)])
        plsc.subcore_barrier()

        def _idx_copy(w, e):
            pltpu.make_async_copy(edges_hbm.at[s * WIN + w], ED[e],
                                  IS[e]).start()

        def _idx_wait(e):
            pltpu.make_async_copy(edges_hbm.at[0], ED[e], IS[e]).wait()

        def _gather_start(e, x):
            pltpu.make_async_copy(xh_hbm.at[h].at[ED[e].at[0]],
                                  RW[x], GS[x]).start()

        def _gather_wait(e, x):
            pltpu.make_async_copy(xh_hbm.at[h].at[ED[e].at[0]],
                                  RW[x], GS[x]).wait()

        def _scat_start(e, x):
            pltpu.make_async_copy(RW[x], num_acc.at[ED[e].at[1]],
                                  SS[x]).start(add=True)
            pltpu.make_async_copy(WB[x], den_acc.at[ED[e].at[1]],
                                  DS[x]).start(add=True)

        def _scat_wait(e, x):
            pltpu.make_async_copy(RW[x], num_acc.at[ED[e].at[1]],
                                  SS[x]).wait()
            pltpu.make_async_copy(WB[x], den_acc.at[ED[e].at[1]],
                                  DS[x]).wait()

        def _alpha(w, e, x):
            g0 = (s * WIN + w) * W
            for i in range(W // 16):
                sv = ED[e][0, pl.ds(i * 16, 16)]
                dv = ED[e][1, pl.ds(i * 16, 16)]
                ps = plsc.unpack(plsc.bitcast(
                    plsc.load_gather(tab_v, [sv]), jnp.bfloat16),
                    format=plsc.PackFormat.INTERLEAVED)[0]
                pd = plsc.unpack(plsc.bitcast(
                    plsc.load_gather(tab_v, [dv]), jnp.bfloat16),
                    format=plsc.PackFormat.INTERLEAVED)[1]
                al = ps + pd
                al = jnp.where(al >= 0, al, al * jnp.float32(0.2))
                wv = jnp.exp(al)
                gi = g0 + i * 16 + lax.iota(jnp.int32, 16)
                wv = jnp.where(gi < E, wv, jnp.float32(0.0))
                WB[x][pl.ds(i * 16, 16)] = wv

        def _scale(x):
            def sbody(gg, _):
                wvec = WB[x][pl.ds(gg * 16, 16)]
                for t in range(16):
                    wsc = wvec[t]
                    r = gg * 16 + t
                    for k in range(8):
                        RW[x][r, pl.ds(k * 16, 16)] = (
                            RW[x][r, pl.ds(k * 16, 16)] * wsc)
                return 0
            lax.fori_loop(0, W // 16, sbody, 0)

        def _step(w, j4, do_g1=True, g1_wait=True, do_s5=True):
            x = j4 % 2
            x1 = (j4 + 1) % 2
            e0 = j4 % 4
            e1 = (j4 + 1) % 4
            e2 = (j4 + 2) % 4
            ep = (j4 + 3) % 4
            if do_g1:
                _idx_wait(e1)
                if g1_wait:
                    _scat_wait(ep, x1)
                _gather_start(e1, x1)
            _alpha(w, e0, x)
            _gather_wait(e0, x)
            _scale(x)
            _scat_start(e0, x)
            if do_s5:
                _idx_copy(w + 2, e2)

        _idx_copy(0, 0)
        _idx_copy(1, 1)
        _idx_wait(0)
        _gather_start(0, 0)
        _step(0, 0, g1_wait=False)
        _step(1, 1)
        _step(2, 2)
        _step(3, 3)

        def _q(q, _):
            w0 = q * 4
            for j4 in range(4):
                _step(w0 + j4, j4)
            return 0
        lax.fori_loop(1, WIN // 4 - 1, _q, 0)

        _step(WIN - 4, 0)
        _step(WIN - 3, 1)
        _step(WIN - 2, 2, do_s5=False)
        _step(WIN - 1, 3, do_g1=False, do_s5=False)
        _scat_wait(2, 0)
        _scat_wait(3, 1)
        plsc.subcore_barrier()

        pltpu.sync_copy(num_acc.at[pl.ds(base_r, RPT)],
                        num_hbm.at[h, pl.ds(base_r, RPT)])
        pltpu.sync_copy(den_acc.at[pl.ds(base_r, RPT)],
                        den_hbm.at[h, pl.ds(base_r, RPT)])
        plsc.subcore_barrier()

        lax.fori_loop(0, W, _zero_rw0, 0)
        return 0
    lax.fori_loop(0, 2, _head, 0)


def _sc_edge_pass(xhT, tab, edges):
    mesh = plsc.VectorSubcoreMesh(core_axis_name="c", subcore_axis_name="s")
    kern = pl.kernel(
        _sc_body,
        out_type=[
            jax.ShapeDtypeStruct((H, N2, DOUT), jnp.float32),
            jax.ShapeDtypeStruct((H, N2), jnp.float32),
        ],
        mesh=mesh,
        compiler_params=pltpu.CompilerParams(needs_layout_passes=False),
        scratch_types=(
            [pltpu.VMEM_SHARED((N2, DOUT), jnp.float32),
             pltpu.VMEM_SHARED((N2,), jnp.float32),
             pltpu.VMEM((N2,), jnp.float32),
             pltpu.VMEM((RPT,), jnp.float32)]
            + [pltpu.VMEM((2, W), jnp.int32)] * 4
            + [pltpu.VMEM((W,), jnp.float32)] * 2
            + [pltpu.VMEM((W, DOUT), jnp.float32)] * 2
            + [pltpu.SemaphoreType.DMA] * 10
        ),
    )
    return kern(xhT, tab, edges)


def _final_body(num_ref, den_ref, xh_ref, as_ref, ad_ref, batch_ref,
                te_ref, wt_ref, bt_ref, bg_ref, lg_ref, lb_ref, out_ref):
    al = as_ref[...] + ad_ref[...]
    al = jnp.where(al >= 0, al, al * 0.2)
    wl = jnp.exp(al)
    inv = 1.0 / (den_ref[...] + wl + 1e-16)
    acc = jnp.zeros((R, DOUT), jnp.float32)
    for h in range(H):
        acc += (num_ref[h] + wl[:, h:h + 1] * xh_ref[h]) * inv[:, h:h + 1]
    hmat = acc * (1.0 / H) + bg_ref[...]
    te = te_ref[...]
    t = jnp.dot(te * jax.nn.sigmoid(te), wt_ref[...],
                preferred_element_type=jnp.float32) + bt_ref[...]
    onehot = (batch_ref[...] ==
              lax.broadcasted_iota(jnp.int32, (1, B), 1)).astype(jnp.float32)
    hmat = hmat + jnp.dot(onehot, t, preferred_element_type=jnp.float32)
    mu = jnp.mean(hmat, axis=1, keepdims=True)
    var = jnp.mean((hmat - mu) ** 2, axis=1, keepdims=True)
    hn = (hmat - mu) * jax.lax.rsqrt(var + 1e-5) * lg_ref[...] + lb_ref[...]
    out_ref[...] = hn * jax.nn.sigmoid(hn)


def _final(num, denT, xhT, asT, adT, batch2, time_emb, W_time, b_time,
           bias_gat, ln_gamma, ln_beta):
    return pl.pallas_call(
        _final_body,
        grid=(N2 // R,),
        in_specs=[
            pl.BlockSpec((H, R, DOUT), lambda r: (0, r, 0)),
            pl.BlockSpec((R, H), lambda r: (r, 0)),
            pl.BlockSpec((H, R, DOUT), lambda r: (0, r, 0)),
            pl.BlockSpec((R, H), lambda r: (r, 0)),
            pl.BlockSpec((R, H), lambda r: (r, 0)),
            pl.BlockSpec((R, 1), lambda r: (r, 0)),
            pl.BlockSpec((B, DTIME), lambda r: (0, 0)),
            pl.BlockSpec((DTIME, DOUT), lambda r: (0, 0)),
            pl.BlockSpec((DOUT,), lambda r: (0,)),
            pl.BlockSpec((DOUT,), lambda r: (0,)),
            pl.BlockSpec((DOUT,), lambda r: (0,)),
            pl.BlockSpec((DOUT,), lambda r: (0,)),
        ],
        out_specs=pl.BlockSpec((R, DOUT), lambda r: (r, 0)),
        out_shape=jax.ShapeDtypeStruct((N2, DOUT), jnp.float32),
    )(num, denT, xhT, asT, adT, batch2, time_emb, W_time, b_time,
      bias_gat, ln_gamma, ln_beta)


@jax.jit
def kernel(x, edge_index, time_emb, batch, W_gat, att_src, att_dst,
           bias_gat, W_time, b_time, ln_gamma, ln_beta):
    x_pad = jnp.pad(x, ((0, N2 - N), (0, 0)))
    srcW = jnp.pad(edge_index[0], (0, EP - E)).reshape(EP // W, W)
    dstW = jnp.pad(edge_index[1], (0, EP - E)).reshape(EP // W, W)
    edges = jnp.stack([srcW, dstW], axis=1)
    batch2 = jnp.pad(batch, (0, N2 - N)).reshape(N2, 1)

    xhT, asrcT, adstT = _prep(x_pad, W_gat,
                              att_src.reshape(H, 1, DOUT),
                              att_dst.reshape(H, 1, DOUT))

    asu = lax.bitcast_convert_type(
        asrcT.astype(jnp.bfloat16), jnp.uint16).astype(jnp.uint32)
    adu = lax.bitcast_convert_type(
        adstT.astype(jnp.bfloat16), jnp.uint16).astype(jnp.uint32)
    tab = lax.bitcast_convert_type(asu | (adu << 16), jnp.float32)

    num, den = _sc_edge_pass(xhT, tab, edges)

    denT = den.T
    asT = asrcT.reshape(H, N2).T
    adT = adstT.reshape(H, N2).T
    out = _final(num, denT, xhT, asT, adT, batch2, time_emb, W_time,
                 b_time, bias_gat, ln_gamma, ln_beta)
    return out[:N]
